# baseline (speedup 1.0000x reference)
import jax
import jax.numpy as jnp
from jax import lax
from jax.experimental import pallas as pl
from jax.experimental.pallas import tpu as pltpu
from jax.experimental.pallas import tpu_sc as plsc

_SEQ = 200
_SEQ_PAD = 208
_DIM = 64
_BATCH = 1024

_NUM_CORES = 2
_NUM_SUBCORES = 16
_NUM_WORKERS = _NUM_CORES * _NUM_SUBCORES
_SEQ_PER_WORKER = _BATCH // _NUM_WORKERS
_CHUNK_A = 104
_CHUNK_B = 96


def _positional_encoding():
    positions = jnp.arange(_SEQ, dtype=jnp.float32)
    indices = jnp.arange(_DIM // 2, dtype=jnp.float32)
    scaling = 10000.0 ** (2.0 * indices / _DIM)
    angles = positions[:, None] / scaling[None, :]
    pe = jnp.zeros((_SEQ, _DIM), dtype=jnp.float32)
    pe = pe.at[:, 0::2].set(jnp.sin(angles))
    pe = pe.at[:, 1::2].set(jnp.cos(angles))
    return jnp.pad(pe, ((0, _SEQ_PAD - _SEQ), (0, 0)))


def _sc_body(x_hbm, table_hbm, pos_hbm, out_hbm, idx_a, idx_b, idx_m, rows_v,
             pos_v, sem):
    wid = lax.axis_index("s") * _NUM_CORES + lax.axis_index("c")
    base = wid * _SEQ_PER_WORKER

    pltpu.sync_copy(pos_hbm, pos_v)
    idx_m[pl.ds(_SEQ, 8)] = jnp.zeros((8,), jnp.int32)

    def per_seq(i, _):
        seq = base + i
        s0 = seq * _SEQ
        pltpu.sync_copy(x_hbm.at[pl.ds(s0, _CHUNK_A)], idx_a)
        pltpu.sync_copy(x_hbm.at[pl.ds(s0 + _CHUNK_A, _CHUNK_B)], idx_b)
        pltpu.sync_copy(x_hbm.at[pl.ds(s0, _SEQ)], idx_m.at[pl.ds(0, _SEQ)])
        g0 = pltpu.async_copy(
            table_hbm.at[idx_a], rows_v.at[pl.ds(0, _CHUNK_A)], sem
        )
        g1 = pltpu.async_copy(
            table_hbm.at[idx_b], rows_v.at[pl.ds(_CHUNK_A, _CHUNK_B)], sem
        )
        g0.wait()
        g1.wait()

        def per_group(h, _):
            t0 = h * 16
            idxg = idx_m[pl.ds(t0, 16)]
            mf = jnp.where(idxg == 0, 0.0, 1.0).astype(jnp.float32)
            for r in range(16):
                m = mf[r]
                t = t0 + r
                for q in range(_DIM // 16):
                    sl = pl.ds(q * 16, 16)
                    rows_v[t, sl] = (rows_v[t, sl] + pos_v[t, sl]) * m
            return 0

        lax.fori_loop(0, _SEQ_PAD // 16, per_group, 0)
        pltpu.sync_copy(rows_v.at[pl.ds(0, _SEQ)], out_hbm.at[seq])
        return 0

    lax.fori_loop(0, _SEQ_PER_WORKER, per_seq, 0)


def kernel(x, table):
    pos = _positional_encoding()
    x = x.astype(jnp.int32).reshape(-1)
    mesh = plsc.VectorSubcoreMesh(core_axis_name="c", subcore_axis_name="s")
    run = pl.kernel(
        _sc_body,
        out_type=jax.ShapeDtypeStruct((_BATCH, _SEQ, _DIM), jnp.float32),
        mesh=mesh,
        scratch_types=[
            pltpu.VMEM((_CHUNK_A,), jnp.int32),
            pltpu.VMEM((_CHUNK_B,), jnp.int32),
            pltpu.VMEM((_SEQ_PAD,), jnp.int32),
            pltpu.VMEM((_SEQ_PAD, _DIM), jnp.float32),
            pltpu.VMEM((_SEQ_PAD, _DIM), jnp.float32),
            pltpu.SemaphoreType.DMA,
        ],
        compiler_params=pltpu.CompilerParams(use_tc_tiling_on_sc=False),
    )
    return run(x, table, pos)
